# baseline (speedup 1.0000x reference)
import functools

import jax
import jax.numpy as jnp
from jax import lax
from jax.experimental import pallas as pl
from jax.experimental.pallas import tpu as pltpu
from jax.experimental.pallas import tpu_sc as plsc

N_NODES = 10000
N_EDGES = 320000
D_EDGE = 16

NC = 2
NS = 16
NW = NC * NS

ROW = 128
N_CHUNKS = N_EDGES // ROW
CPT = N_CHUNKS // NW
REM_CHUNKS = N_CHUNKS - CPT * NW
K = 13
STEPS = CPT // K
CHUNK = K * ROW
N_PAD = 10240
STRIPE = N_PAD // NS


TR_UNROLL = 8


def _sc_body(
    zeros_hbm, idx_hbm, attr_t_hbm, out_hbm, idx_v, attr_tv, attr_tr, attr_v, acc
):
    _IOTA16 = lax.iota(jnp.int32, 16)
    _ZERO16 = jnp.zeros((16,), jnp.int32)
    cid = lax.axis_index("c")
    sid = lax.axis_index("s")
    wid = sid * NC + cid

    stripe = pl.ds(sid * STRIPE, STRIPE)
    pltpu.sync_copy(zeros_hbm.at[stripe], acc.at[stripe])
    plsc.subcore_barrier()

    c0 = wid * CPT

    def step(s, carry):
        cbase = c0 + s * K
        pltpu.sync_copy(idx_hbm.at[pl.ds(cbase, K), 1], idx_v)
        pltpu.sync_copy(attr_t_hbm.at[:, pl.ds(cbase * ROW, CHUNK)], attr_tv)

        def trans(g, carry2):
            for u in range(TR_UNROLL):
                e = g * TR_UNROLL + u
                row = plsc.load_gather(attr_tv, [_IOTA16, _ZERO16 + e])
                attr_v[e, :] = row
            return carry2

        lax.fori_loop(0, CHUNK // TR_UNROLL, trans, 0)
        for j in range(K):
            pltpu.sync_copy(
                attr_v.at[pl.ds(j * ROW, ROW)],
                acc.at[idx_v.at[j]],
                add=True,
            )
        return carry

    lax.fori_loop(0, STEPS, step, 0)

    @pl.when(wid < REM_CHUNKS)
    def _rem_chunk():
        c = CPT * NW + wid
        pltpu.sync_copy(idx_hbm.at[pl.ds(c, 1), 1], idx_v.at[pl.ds(0, 1)])
        pltpu.sync_copy(attr_t_hbm.at[:, pl.ds(c * ROW, ROW)], attr_tr)

        def trans_rem(g, carry2):
            for u in range(TR_UNROLL):
                e = g * TR_UNROLL + u
                row = plsc.load_gather(attr_tr, [_IOTA16, _ZERO16 + e])
                attr_v[e, :] = row
            return carry2

        lax.fori_loop(0, ROW // TR_UNROLL, trans_rem, 0)
        pltpu.sync_copy(
            attr_v.at[pl.ds(0, ROW)],
            acc.at[idx_v.at[0]],
            add=True,
        )

    plsc.subcore_barrier()

    pltpu.sync_copy(acc.at[stripe], out_hbm.at[cid, stripe])


_sc_scatter = functools.partial(
    pl.kernel,
    mesh=plsc.VectorSubcoreMesh(core_axis_name="c", subcore_axis_name="s"),
    out_type=jax.ShapeDtypeStruct((NC, N_PAD, D_EDGE), jnp.float32),
    scratch_types=[
        pltpu.VMEM((K, ROW), jnp.int32),
        pltpu.VMEM((D_EDGE, CHUNK), jnp.float32),
        pltpu.VMEM((D_EDGE, ROW), jnp.float32),
        pltpu.VMEM((CHUNK, D_EDGE), jnp.float32),
        pltpu.VMEM_SHARED((N_PAD, D_EDGE), jnp.float32),
    ],
    compiler_params=pltpu.CompilerParams(
        use_tc_tiling_on_sc=False, needs_layout_passes=False
    ),
)(_sc_body)


def _combine_body(p_ref, o_ref):
    o_ref[...] = p_ref[0, :N_NODES] + p_ref[1, :N_NODES]


@jax.jit
def kernel(x, edge_index, edge_attr):
    del x
    attr_t = edge_attr.T
    idx3 = jnp.transpose(
        edge_index.astype(jnp.int32).reshape(2, N_CHUNKS, ROW), (1, 0, 2)
    )
    zeros = jnp.zeros((N_PAD, D_EDGE), jnp.float32)
    partials = _sc_scatter(zeros, idx3, attr_t)
    return pl.pallas_call(
        _combine_body,
        out_shape=jax.ShapeDtypeStruct((N_NODES, D_EDGE), jnp.float32),
    )(partials)


# speedup vs baseline: 1.6614x; 1.6614x over previous
import functools

import jax
import jax.numpy as jnp
from jax import lax
from jax.experimental import pallas as pl
from jax.experimental.pallas import tpu as pltpu
from jax.experimental.pallas import tpu_sc as plsc

N_NODES = 10000
N_EDGES = 320000
D_EDGE = 16

NC = 2
NS = 16
NW = NC * NS

ROW = 128
N_CHUNKS = N_EDGES // ROW
CPT = N_CHUNKS // NW
REM_CHUNKS = N_CHUNKS - CPT * NW
K = 13
STEPS = CPT // K
CHUNK = K * ROW
N_PAD = 10240
STRIPE = N_PAD // NS


def _sc_body(
    zeros_hbm, idx_hbm, attr_t_hbm, out_hbm, idx_v, attr_tv, attr_tr, attr_v, acc
):
    _IOTA16 = lax.iota(jnp.int32, 16)
    cid = lax.axis_index("c")
    sid = lax.axis_index("s")
    wid = sid * NC + cid

    stripe = pl.ds(sid * STRIPE, STRIPE)
    pltpu.sync_copy(zeros_hbm.at[stripe], acc.at[stripe])
    plsc.subcore_barrier()

    c0 = wid * CPT

    def step(s, carry):
        cbase = c0 + s * K
        pltpu.sync_copy(idx_hbm.at[1, pl.ds(cbase * ROW, CHUNK)], idx_v)
        pltpu.sync_copy(attr_t_hbm.at[:, pl.ds(cbase * ROW, CHUNK)], attr_tv)

        def trans(g, carry2):
            rows = _IOTA16 + g * 16
            for f in range(D_EDGE):
                vals = attr_tv[f, pl.ds(g * 16, 16)]
                plsc.store_scatter(attr_v, [rows, _IOTA16 * 0 + f], vals)
            return carry2

        lax.fori_loop(0, CHUNK // 16, trans, 0)
        for j in range(K):
            pltpu.sync_copy(
                attr_v.at[pl.ds(j * ROW, ROW)],
                acc.at[idx_v.at[pl.ds(j * ROW, ROW)]],
                add=True,
            )
        return carry

    lax.fori_loop(0, STEPS, step, 0)

    @pl.when(wid < REM_CHUNKS)
    def _rem_chunk():
        c = CPT * NW + wid
        pltpu.sync_copy(idx_hbm.at[1, pl.ds(c * ROW, ROW)], idx_v.at[pl.ds(0, ROW)])
        pltpu.sync_copy(attr_t_hbm.at[:, pl.ds(c * ROW, ROW)], attr_tr)

        def trans_rem(g, carry2):
            rows = _IOTA16 + g * 16
            for f in range(D_EDGE):
                vals = attr_tr[f, pl.ds(g * 16, 16)]
                plsc.store_scatter(attr_v, [rows, _IOTA16 * 0 + f], vals)
            return carry2

        lax.fori_loop(0, ROW // 16, trans_rem, 0)
        pltpu.sync_copy(
            attr_v.at[pl.ds(0, ROW)],
            acc.at[idx_v.at[pl.ds(0, ROW)]],
            add=True,
        )

    plsc.subcore_barrier()

    pltpu.sync_copy(acc.at[stripe], out_hbm.at[cid, stripe])


_sc_scatter = functools.partial(
    pl.kernel,
    mesh=plsc.VectorSubcoreMesh(core_axis_name="c", subcore_axis_name="s"),
    out_type=jax.ShapeDtypeStruct((NC, N_PAD, D_EDGE), jnp.float32),
    scratch_types=[
        pltpu.VMEM((CHUNK,), jnp.int32),
        pltpu.VMEM((D_EDGE, CHUNK), jnp.float32),
        pltpu.VMEM((D_EDGE, ROW), jnp.float32),
        pltpu.VMEM((CHUNK, D_EDGE), jnp.float32),
        pltpu.VMEM_SHARED((N_PAD, D_EDGE), jnp.float32),
    ],
    compiler_params=pltpu.CompilerParams(
        use_tc_tiling_on_sc=False, needs_layout_passes=False
    ),
)(_sc_body)


def _combine_body(p_ref, o_ref):
    o_ref[...] = p_ref[0, :N_NODES] + p_ref[1, :N_NODES]


@jax.jit
def kernel(x, edge_index, edge_attr):
    del x
    attr_t = edge_attr.T
    zeros = jnp.zeros((N_PAD, D_EDGE), jnp.float32)
    partials = _sc_scatter(zeros, edge_index.astype(jnp.int32), attr_t)
    return pl.pallas_call(
        _combine_body,
        out_shape=jax.ShapeDtypeStruct((N_NODES, D_EDGE), jnp.float32),
    )(partials)


# speedup vs baseline: 1.9269x; 1.1598x over previous
import functools

import jax
import jax.numpy as jnp
from jax import lax
from jax.experimental import pallas as pl
from jax.experimental.pallas import tpu as pltpu
from jax.experimental.pallas import tpu_sc as plsc

N_NODES = 10000
N_EDGES = 320000
D_EDGE = 16

NC = 2
NS = 16
NW = NC * NS

ROW = 128
N_CHUNKS = N_EDGES // ROW
CPT = N_CHUNKS // NW
REM_CHUNKS = N_CHUNKS - CPT * NW
K = 13
STEPS = CPT // K
CHUNK = K * ROW
N_PAD = 10240
STRIPE = N_PAD // NS


def _sc_body(
    zeros_hbm, idx_hbm, attr4_hbm, out_hbm, idx_v, attr_s, attr_r, attr_v, acc
):
    _IOTA16 = lax.iota(jnp.int32, 16)
    _ROWS0 = [_IOTA16 + gg * 16 for gg in range(8)]
    _COLS = [_IOTA16 * 0 + f for f in range(D_EDGE)]
    cid = lax.axis_index("c")
    sid = lax.axis_index("s")
    wid = sid * NC + cid

    stripe = pl.ds(sid * STRIPE, STRIPE)
    pltpu.sync_copy(zeros_hbm.at[stripe], acc.at[stripe])
    plsc.subcore_barrier()

    c0 = wid * CPT

    def step(s, carry):
        cbase = c0 + s * K
        pltpu.sync_copy(idx_hbm.at[1, pl.ds(cbase * ROW, CHUNK)], idx_v)
        pltpu.sync_copy(attr4_hbm.at[:, pl.ds(cbase, K)], attr_s)

        def trans(j, carry2):
            jrow = j * 128
            for gg in range(8):
                rows = _ROWS0[gg] + jrow
                for f in range(D_EDGE):
                    vals = attr_s[f // 8, j, f % 8, pl.ds(gg * 16, 16)]
                    plsc.store_scatter(attr_v, [rows, _COLS[f]], vals)
            return carry2

        lax.fori_loop(0, K, trans, 0)
        for j in range(K):
            pltpu.sync_copy(
                attr_v.at[pl.ds(j * ROW, ROW)],
                acc.at[idx_v.at[pl.ds(j * ROW, ROW)]],
                add=True,
            )
        return carry

    lax.fori_loop(0, STEPS, step, 0)

    @pl.when(wid < REM_CHUNKS)
    def _rem_chunk():
        c = CPT * NW + wid
        pltpu.sync_copy(idx_hbm.at[1, pl.ds(c * ROW, ROW)], idx_v.at[pl.ds(0, ROW)])
        pltpu.sync_copy(attr4_hbm.at[:, pl.ds(c, 1)], attr_r)
        for gg in range(8):
            for f in range(D_EDGE):
                vals = attr_r[f // 8, 0, f % 8, pl.ds(gg * 16, 16)]
                plsc.store_scatter(attr_v, [_ROWS0[gg], _COLS[f]], vals)
        pltpu.sync_copy(
            attr_v.at[pl.ds(0, ROW)],
            acc.at[idx_v.at[pl.ds(0, ROW)]],
            add=True,
        )

    plsc.subcore_barrier()

    pltpu.sync_copy(acc.at[stripe], out_hbm.at[cid, stripe])


_sc_scatter = functools.partial(
    pl.kernel,
    mesh=plsc.VectorSubcoreMesh(core_axis_name="c", subcore_axis_name="s"),
    out_type=jax.ShapeDtypeStruct((NC, N_PAD, D_EDGE), jnp.float32),
    scratch_types=[
        pltpu.VMEM((CHUNK,), jnp.int32),
        pltpu.VMEM((2, K, 8, ROW), jnp.float32),
        pltpu.VMEM((2, 1, 8, ROW), jnp.float32),
        pltpu.VMEM((CHUNK, D_EDGE), jnp.float32),
        pltpu.VMEM_SHARED((N_PAD, D_EDGE), jnp.float32),
    ],
    compiler_params=pltpu.CompilerParams(
        use_tc_tiling_on_sc=False, needs_layout_passes=False
    ),
)(_sc_body)


def _combine_body(p_ref, o_ref):
    o_ref[...] = p_ref[0, :N_NODES] + p_ref[1, :N_NODES]


@jax.jit
def kernel(x, edge_index, edge_attr):
    del x
    attr4 = edge_attr.T.reshape(2, 8, N_CHUNKS, ROW).transpose(0, 2, 1, 3)
    zeros = jnp.zeros((N_PAD, D_EDGE), jnp.float32)
    partials = _sc_scatter(zeros, edge_index.astype(jnp.int32), attr4)
    return pl.pallas_call(
        _combine_body,
        out_shape=jax.ShapeDtypeStruct((N_NODES, D_EDGE), jnp.float32),
    )(partials)


# speedup vs baseline: 2.0805x; 1.0797x over previous
import functools

import jax
import jax.numpy as jnp
from jax import lax
from jax.experimental import pallas as pl
from jax.experimental.pallas import tpu as pltpu
from jax.experimental.pallas import tpu_sc as plsc

N_NODES = 10000
N_EDGES = 320000
D_EDGE = 16

NC = 2
NS = 16
NW = NC * NS

ROW = 128
N_CHUNKS = N_EDGES // ROW
CPT = N_CHUNKS // NW
REM_CHUNKS = N_CHUNKS - CPT * NW
K = 13
STEPS = CPT // K
CHUNK = K * ROW
N_PAD = 10240
STRIPE = N_PAD // NS


def _sc_body(
    zeros_hbm, idx_hbm, attr4_hbm, out_hbm, idx_v, attr_s, attr_r, attr_v, acc, sem
):
    _IOTA16 = lax.iota(jnp.int32, 16)
    _ROWS0 = [_IOTA16 + gg * 16 for gg in range(8)]
    _COLS = [_IOTA16 * 0 + f for f in range(D_EDGE)]
    cid = lax.axis_index("c")
    sid = lax.axis_index("s")
    wid = sid * NC + cid

    stripe = pl.ds(sid * STRIPE, STRIPE)
    pltpu.sync_copy(zeros_hbm.at[stripe], acc.at[stripe])
    plsc.subcore_barrier()

    c0 = wid * CPT

    def stage_and_transpose(s, slot):
        cbase = c0 + s * K
        pltpu.sync_copy(
            idx_hbm.at[1, pl.ds(cbase * ROW, CHUNK)], idx_v.at[slot]
        )
        pltpu.sync_copy(attr4_hbm.at[:, pl.ds(cbase, K)], attr_s)

        def trans(j, carry2):
            jrow = j * 128
            for gg in range(8):
                rows = _ROWS0[gg] + jrow
                for f in range(D_EDGE):
                    vals = attr_s[f // 8, j, f % 8, pl.ds(gg * 16, 16)]
                    plsc.store_scatter(attr_v.at[slot], [rows, _COLS[f]], vals)
            return carry2

        lax.fori_loop(0, K, trans, 0)

    stage_and_transpose(0, 0)
    for s in range(STEPS):
        cur = s % 2
        descs = [
            pltpu.async_copy(
                attr_v.at[cur, pl.ds(j * ROW, ROW)],
                acc.at[idx_v.at[cur, pl.ds(j * ROW, ROW)]],
                sem,
                add=True,
            )
            for j in range(K)
        ]
        if s + 1 < STEPS:
            stage_and_transpose(s + 1, 1 - cur)
        for d in descs:
            d.wait()

    @pl.when(wid < REM_CHUNKS)
    def _rem_chunk():
        c = CPT * NW + wid
        pltpu.sync_copy(
            idx_hbm.at[1, pl.ds(c * ROW, ROW)], idx_v.at[0, pl.ds(0, ROW)]
        )
        pltpu.sync_copy(attr4_hbm.at[:, pl.ds(c, 1)], attr_r)
        for gg in range(8):
            for f in range(D_EDGE):
                vals = attr_r[f // 8, 0, f % 8, pl.ds(gg * 16, 16)]
                plsc.store_scatter(attr_v.at[0], [_ROWS0[gg], _COLS[f]], vals)
        pltpu.sync_copy(
            attr_v.at[0, pl.ds(0, ROW)],
            acc.at[idx_v.at[0, pl.ds(0, ROW)]],
            add=True,
        )

    plsc.subcore_barrier()

    pltpu.sync_copy(acc.at[stripe], out_hbm.at[cid, stripe])


_sc_scatter = functools.partial(
    pl.kernel,
    mesh=plsc.VectorSubcoreMesh(core_axis_name="c", subcore_axis_name="s"),
    out_type=jax.ShapeDtypeStruct((NC, N_PAD, D_EDGE), jnp.float32),
    scratch_types=[
        pltpu.VMEM((2, CHUNK), jnp.int32),
        pltpu.VMEM((2, K, 8, ROW), jnp.float32),
        pltpu.VMEM((2, 1, 8, ROW), jnp.float32),
        pltpu.VMEM((2, CHUNK, D_EDGE), jnp.float32),
        pltpu.VMEM_SHARED((N_PAD, D_EDGE), jnp.float32),
        pltpu.SemaphoreType.DMA,
    ],
    compiler_params=pltpu.CompilerParams(
        use_tc_tiling_on_sc=False, needs_layout_passes=False
    ),
)(_sc_body)


def _combine_body(p_ref, o_ref):
    o_ref[...] = p_ref[0, :N_NODES] + p_ref[1, :N_NODES]


@jax.jit
def kernel(x, edge_index, edge_attr):
    del x
    attr4 = edge_attr.T.reshape(2, 8, N_CHUNKS, ROW).transpose(0, 2, 1, 3)
    zeros = jnp.zeros((N_PAD, D_EDGE), jnp.float32)
    partials = _sc_scatter(zeros, edge_index.astype(jnp.int32), attr4)
    return pl.pallas_call(
        _combine_body,
        out_shape=jax.ShapeDtypeStruct((N_NODES, D_EDGE), jnp.float32),
    )(partials)


# speedup vs baseline: 2.3819x; 1.1448x over previous
import functools

import jax
import jax.numpy as jnp
from jax import lax
from jax.experimental import pallas as pl
from jax.experimental.pallas import tpu as pltpu
from jax.experimental.pallas import tpu_sc as plsc

N_NODES = 10000
N_EDGES = 320000
D_EDGE = 16

NC = 2
NS = 16
NW = NC * NS

ROW = 128
N_CHUNKS = N_EDGES // ROW
CPT = N_CHUNKS // NW
REM_CHUNKS = N_CHUNKS - CPT * NW
K = 13
STEPS = CPT // K
CHUNK = K * ROW
N_PAD = 10240
STRIPE = N_PAD // NS


def _sc_body(
    zeros_hbm,
    idx_hbm,
    attr4_hbm,
    out_hbm,
    idx_v,
    attr_s,
    attr_r,
    attr_v,
    acc,
    sem,
    sem_in,
):
    _IOTA16 = lax.iota(jnp.int32, 16)
    _ROWS0 = [_IOTA16 + gg * 16 for gg in range(8)]
    _COLS = [_IOTA16 * 0 + f for f in range(D_EDGE)]
    cid = lax.axis_index("c")
    sid = lax.axis_index("s")
    wid = sid * NC + cid

    stripe = pl.ds(sid * STRIPE, STRIPE)
    pltpu.sync_copy(zeros_hbm.at[stripe], acc.at[stripe])
    plsc.subcore_barrier()

    c0 = wid * CPT

    def stage_start(s):
        cbase = c0 + s * K
        return [
            pltpu.async_copy(
                idx_hbm.at[1, pl.ds(cbase * ROW, CHUNK)], idx_v.at[s % 3], sem_in
            ),
            pltpu.async_copy(
                attr4_hbm.at[:, pl.ds(cbase, K)], attr_s.at[s % 2], sem_in
            ),
        ]

    def transpose(s):
        sslot = s % 2

        def trans(j, carry2):
            jrow = j * 128
            for gg in range(8):
                rows = _ROWS0[gg] + jrow
                for f in range(D_EDGE):
                    vals = attr_s[sslot, f // 8, j, f % 8, pl.ds(gg * 16, 16)]
                    plsc.store_scatter(attr_v.at[sslot], [rows, _COLS[f]], vals)
            return carry2

        lax.fori_loop(0, K, trans, 0)

    stage = stage_start(0)
    for d in stage:
        d.wait()
    stage = stage_start(1)
    transpose(0)
    for s in range(STEPS):
        cur = s % 2
        descs = [
            pltpu.async_copy(
                attr_v.at[cur, pl.ds(j * ROW, ROW)],
                acc.at[idx_v.at[s % 3, pl.ds(j * ROW, ROW)]],
                sem,
                add=True,
            )
            for j in range(K)
        ]
        if s + 1 < STEPS:
            for d in stage:
                d.wait()
            if s + 2 < STEPS:
                stage = stage_start(s + 2)
            transpose(s + 1)
        for d in descs:
            d.wait()

    @pl.when(wid < REM_CHUNKS)
    def _rem_chunk():
        c = CPT * NW + wid
        pltpu.sync_copy(
            idx_hbm.at[1, pl.ds(c * ROW, ROW)], idx_v.at[0, pl.ds(0, ROW)]
        )
        pltpu.sync_copy(attr4_hbm.at[:, pl.ds(c, 1)], attr_r)
        for gg in range(8):
            for f in range(D_EDGE):
                vals = attr_r[f // 8, 0, f % 8, pl.ds(gg * 16, 16)]
                plsc.store_scatter(attr_v.at[0], [_ROWS0[gg], _COLS[f]], vals)
        pltpu.sync_copy(
            attr_v.at[0, pl.ds(0, ROW)],
            acc.at[idx_v.at[0, pl.ds(0, ROW)]],
            add=True,
        )

    plsc.subcore_barrier()

    pltpu.sync_copy(acc.at[stripe], out_hbm.at[cid, stripe])


_sc_scatter = functools.partial(
    pl.kernel,
    mesh=plsc.VectorSubcoreMesh(core_axis_name="c", subcore_axis_name="s"),
    out_type=jax.ShapeDtypeStruct((NC, N_PAD, D_EDGE), jnp.float32),
    scratch_types=[
        pltpu.VMEM((3, CHUNK), jnp.int32),
        pltpu.VMEM((2, 2, K, 8, ROW), jnp.float32),
        pltpu.VMEM((2, 1, 8, ROW), jnp.float32),
        pltpu.VMEM((2, CHUNK, D_EDGE), jnp.float32),
        pltpu.VMEM_SHARED((N_PAD, D_EDGE), jnp.float32),
        pltpu.SemaphoreType.DMA,
        pltpu.SemaphoreType.DMA,
    ],
    compiler_params=pltpu.CompilerParams(
        use_tc_tiling_on_sc=False, needs_layout_passes=False
    ),
)(_sc_body)


def _combine_body(p_ref, o_ref):
    o_ref[...] = p_ref[0, :N_NODES] + p_ref[1, :N_NODES]


@jax.jit
def kernel(x, edge_index, edge_attr):
    del x
    attr4 = edge_attr.T.reshape(2, 8, N_CHUNKS, ROW).transpose(0, 2, 1, 3)
    zeros = jnp.zeros((N_PAD, D_EDGE), jnp.float32)
    partials = _sc_scatter(zeros, edge_index.astype(jnp.int32), attr4)
    return pl.pallas_call(
        _combine_body,
        out_shape=jax.ShapeDtypeStruct((N_NODES, D_EDGE), jnp.float32),
    )(partials)


# speedup vs baseline: 2.6321x; 1.1051x over previous
import functools

import jax
import jax.numpy as jnp
from jax import lax
from jax.experimental import pallas as pl
from jax.experimental.pallas import tpu as pltpu
from jax.experimental.pallas import tpu_sc as plsc

N_NODES = 10000
N_EDGES = 320000
D_EDGE = 16

NC = 2
NS = 16
NW = NC * NS

ROW = 128
N_CHUNKS = N_EDGES // ROW
CPT = N_CHUNKS // NW
REM_CHUNKS = N_CHUNKS - CPT * NW
K = 13
STEPS = CPT // K
CHUNK = K * ROW
N_PAD = 10240
STRIPE = N_PAD // NS


def _sc_body(
    zeros_hbm,
    idx_hbm,
    attr4_hbm,
    out_hbm,
    idx_v,
    attr_s,
    attr_r,
    attr_v,
    acc,
    sem,
    sem_in,
):
    _IOTA16 = lax.iota(jnp.int32, 16)
    _ROWS0 = [_IOTA16 + gg * 16 for gg in range(8)]
    _COLS = [_IOTA16 * 0 + f for f in range(D_EDGE)]
    cid = lax.axis_index("c")
    sid = lax.axis_index("s")
    wid = sid * NC + cid

    stripe = pl.ds(sid * STRIPE, STRIPE)
    pltpu.sync_copy(zeros_hbm.at[stripe], acc.at[stripe])
    plsc.subcore_barrier()

    c0 = wid * CPT

    def stage_start(s):
        cbase = c0 + s * K
        return [
            pltpu.async_copy(
                idx_hbm.at[1, pl.ds(cbase * ROW, CHUNK)], idx_v.at[s % 3], sem_in
            ),
            pltpu.async_copy(
                attr4_hbm.at[:, pl.ds(cbase, K)], attr_s.at[s % 2], sem_in
            ),
        ]

    def transpose(s):
        sslot = s % 2

        def trans(j, carry2):
            jrow = j * 128
            for gg in range(8):
                rows = _ROWS0[gg] + jrow
                for f in range(D_EDGE):
                    vals = attr_s[sslot, f // 8, j, f % 8, pl.ds(gg * 16, 16)]
                    plsc.store_scatter(attr_v.at[sslot], [rows, _COLS[f]], vals)
            return carry2

        lax.fori_loop(0, K, trans, 0)

    stage = stage_start(0)
    for d in stage:
        d.wait()
    stage = stage_start(1)
    transpose(0)
    for s in range(STEPS):
        cur = s % 2
        descs = [
            pltpu.async_copy(
                attr_v.at[cur, pl.ds(j * ROW, ROW)],
                acc.at[idx_v.at[s % 3, pl.ds(j * ROW, ROW)]],
                sem,
                add=True,
            )
            for j in range(K)
        ]
        if s + 1 < STEPS:
            for d in stage:
                d.wait()
            if s + 2 < STEPS:
                stage = stage_start(s + 2)
            transpose(s + 1)
        for d in descs:
            d.wait()

    @pl.when(wid < REM_CHUNKS)
    def _rem_chunk():
        c = CPT * NW + wid
        pltpu.sync_copy(
            idx_hbm.at[1, pl.ds(c * ROW, ROW)], idx_v.at[0, pl.ds(0, ROW)]
        )
        pltpu.sync_copy(attr4_hbm.at[:, pl.ds(c, 1)], attr_r)
        for gg in range(8):
            for f in range(D_EDGE):
                vals = attr_r[f // 8, 0, f % 8, pl.ds(gg * 16, 16)]
                plsc.store_scatter(attr_v.at[0], [_ROWS0[gg], _COLS[f]], vals)
        pltpu.sync_copy(
            attr_v.at[0, pl.ds(0, ROW)],
            acc.at[idx_v.at[0, pl.ds(0, ROW)]],
            add=True,
        )

    plsc.subcore_barrier()

    pltpu.sync_copy(acc.at[stripe], out_hbm.at[cid, stripe])


_sc_scatter = functools.partial(
    pl.kernel,
    mesh=plsc.VectorSubcoreMesh(core_axis_name="c", subcore_axis_name="s"),
    out_type=jax.ShapeDtypeStruct((NC, N_PAD, D_EDGE), jnp.float32),
    scratch_types=[
        pltpu.VMEM((3, CHUNK), jnp.int32),
        pltpu.VMEM((2, 2, K, 8, ROW), jnp.float32),
        pltpu.VMEM((2, 1, 8, ROW), jnp.float32),
        pltpu.VMEM((2, CHUNK, D_EDGE), jnp.float32),
        pltpu.VMEM_SHARED((N_PAD, D_EDGE), jnp.float32),
        pltpu.SemaphoreType.DMA,
        pltpu.SemaphoreType.DMA,
    ],
    compiler_params=pltpu.CompilerParams(
        use_tc_tiling_on_sc=False, needs_layout_passes=False
    ),
)(_sc_body)


def _combine_body(p_ref, o_ref):
    o_ref[...] = p_ref[0] + p_ref[1]


@jax.jit
def kernel(x, edge_index, edge_attr):
    del x
    attr4 = edge_attr.T.reshape(2, 8, N_CHUNKS, ROW).transpose(0, 2, 1, 3)
    zeros = jnp.zeros((N_PAD, D_EDGE), jnp.float32)
    partials = _sc_scatter(zeros, edge_index.astype(jnp.int32), attr4)
    wide = N_PAD * D_EDGE // 128
    combined = pl.pallas_call(
        _combine_body,
        out_shape=jax.ShapeDtypeStruct((wide, 128), jnp.float32),
    )(partials.reshape(NC, wide, 128))
    return combined.reshape(N_PAD, D_EDGE)[:N_NODES]


# speedup vs baseline: 2.6647x; 1.0124x over previous
import functools

import jax
import jax.numpy as jnp
from jax import lax
from jax.experimental import pallas as pl
from jax.experimental.pallas import tpu as pltpu
from jax.experimental.pallas import tpu_sc as plsc

N_NODES = 10000
N_EDGES = 320000
D_EDGE = 16

NC = 2
NS = 16
NW = NC * NS

ROW = 128
N_CHUNKS = N_EDGES // ROW
CPT = N_CHUNKS // NW
REM_CHUNKS = N_CHUNKS - CPT * NW
K = 13
STEPS = CPT // K
CHUNK = K * ROW
N_PAD = 10240
STRIPE = N_PAD // NS


def _sc_body(
    zeros_hbm,
    idx_hbm,
    attr4_hbm,
    out_hbm,
    idx_v,
    attr_s,
    attr_r,
    attr_v,
    acc,
    sem,
    sem_in,
):
    _IOTA16 = lax.iota(jnp.int32, 16)
    _ROWS0 = [_IOTA16 + gg * 16 for gg in range(8)]
    _COLS = [_IOTA16 * 0 + f for f in range(D_EDGE)]
    cid = lax.axis_index("c")
    sid = lax.axis_index("s")
    wid = sid * NC + cid

    stripe = pl.ds(sid * STRIPE, STRIPE)
    pltpu.sync_copy(zeros_hbm.at[stripe], acc.at[stripe])
    plsc.subcore_barrier()

    c0 = wid * CPT

    def stage_start(s):
        cbase = c0 + s * K
        return [
            pltpu.async_copy(
                idx_hbm.at[pl.ds(cbase, K), 1], idx_v.at[s % 3], sem_in
            ),
            pltpu.async_copy(
                attr4_hbm.at[:, pl.ds(cbase, K)], attr_s.at[s % 2], sem_in
            ),
        ]

    def transpose(s):
        sslot = s % 2

        def trans(j, carry2):
            jrow = j * 128
            for gg in range(8):
                rows = _ROWS0[gg] + jrow
                for f in range(D_EDGE):
                    vals = attr_s[sslot, f // 8, j, f % 8, pl.ds(gg * 16, 16)]
                    plsc.store_scatter(attr_v.at[sslot], [rows, _COLS[f]], vals)
            return carry2

        lax.fori_loop(0, K, trans, 0)

    stage = stage_start(0)
    for d in stage:
        d.wait()
    stage = stage_start(1)
    transpose(0)
    for s in range(STEPS):
        cur = s % 2
        descs = [
            pltpu.async_copy(
                attr_v.at[cur, pl.ds(j * ROW, ROW)],
                acc.at[idx_v.at[s % 3, j]],
                sem,
                add=True,
            )
            for j in range(K)
        ]
        if s + 1 < STEPS:
            for d in stage:
                d.wait()
            if s + 2 < STEPS:
                stage = stage_start(s + 2)
            transpose(s + 1)
        for d in descs:
            d.wait()

    @pl.when(wid < REM_CHUNKS)
    def _rem_chunk():
        c = CPT * NW + wid
        pltpu.sync_copy(idx_hbm.at[pl.ds(c, 1), 1], idx_v.at[0, pl.ds(0, 1)])
        pltpu.sync_copy(attr4_hbm.at[:, pl.ds(c, 1)], attr_r)
        for gg in range(8):
            for f in range(D_EDGE):
                vals = attr_r[f // 8, 0, f % 8, pl.ds(gg * 16, 16)]
                plsc.store_scatter(attr_v.at[0], [_ROWS0[gg], _COLS[f]], vals)
        pltpu.sync_copy(
            attr_v.at[0, pl.ds(0, ROW)],
            acc.at[idx_v.at[0, 0]],
            add=True,
        )

    plsc.subcore_barrier()

    pltpu.sync_copy(acc.at[stripe], out_hbm.at[cid, stripe])


_sc_scatter = functools.partial(
    pl.kernel,
    mesh=plsc.VectorSubcoreMesh(core_axis_name="c", subcore_axis_name="s"),
    out_type=jax.ShapeDtypeStruct((NC, N_PAD, D_EDGE), jnp.float32),
    scratch_types=[
        pltpu.VMEM((3, K, ROW), jnp.int32),
        pltpu.VMEM((2, 2, K, 8, ROW), jnp.float32),
        pltpu.VMEM((2, 1, 8, ROW), jnp.float32),
        pltpu.VMEM((2, CHUNK, D_EDGE), jnp.float32),
        pltpu.VMEM_SHARED((N_PAD, D_EDGE), jnp.float32),
        pltpu.SemaphoreType.DMA,
        pltpu.SemaphoreType.DMA,
    ],
    compiler_params=pltpu.CompilerParams(
        use_tc_tiling_on_sc=False, needs_layout_passes=False
    ),
)(_sc_body)


def _combine_body(p_ref, o_ref):
    o_ref[...] = p_ref[0] + p_ref[1]


@jax.jit
def kernel(x, edge_index, edge_attr):
    del x
    attr4 = edge_attr.T.reshape(2, 8, N_CHUNKS, ROW).transpose(0, 2, 1, 3)
    idx3 = edge_index.astype(jnp.int32).reshape(2, N_CHUNKS, ROW).transpose(1, 0, 2)
    zeros = jnp.zeros((N_PAD, D_EDGE), jnp.float32)
    partials = _sc_scatter(zeros, idx3, attr4)
    wide = N_PAD * D_EDGE // 128
    combined = pl.pallas_call(
        _combine_body,
        out_shape=jax.ShapeDtypeStruct((wide, 128), jnp.float32),
    )(partials.reshape(NC, wide, 128))
    return combined.reshape(N_PAD, D_EDGE)[:N_NODES]


# speedup vs baseline: 2.6682x; 1.0013x over previous
import functools

import jax
import jax.numpy as jnp
from jax import lax
from jax.experimental import pallas as pl
from jax.experimental.pallas import tpu as pltpu
from jax.experimental.pallas import tpu_sc as plsc

N_NODES = 10000
N_EDGES = 320000
D_EDGE = 16

NC = 2
NS = 16
NW = NC * NS

ROW = 128
N_CHUNKS = N_EDGES // ROW
CPT = N_CHUNKS // NW
REM_CHUNKS = N_CHUNKS - CPT * NW
K = 13
STEPS = CPT // K
CHUNK = K * ROW
N_PAD = 10240
STRIPE = N_PAD // NS


def _sc_body(
    zeros_hbm,
    idx_hbm,
    attr4_hbm,
    out_hbm,
    idx_v,
    attr_s,
    attr_r,
    attr_v,
    acc,
    sem,
    sem_in,
):
    _IOTA16 = lax.iota(jnp.int32, 16)
    _ROWS0 = [_IOTA16 + gg * 16 for gg in range(8)]
    _COLS = [_IOTA16 * 0 + f for f in range(D_EDGE)]
    cid = lax.axis_index("c")
    sid = lax.axis_index("s")
    wid = sid * NC + cid

    stripe = pl.ds(sid * STRIPE, STRIPE)
    pltpu.sync_copy(zeros_hbm.at[stripe], acc.at[stripe])
    plsc.subcore_barrier()

    c0 = wid * CPT

    def stage_start(s):
        cbase = c0 + s * K
        return [
            pltpu.async_copy(
                idx_hbm.at[pl.ds(cbase, K), 1], idx_v.at[s % 3], sem_in
            ),
            pltpu.async_copy(
                attr4_hbm.at[:, pl.ds(cbase, K)], attr_s.at[s % 2], sem_in
            ),
        ]

    def transpose(s):
        sslot = s % 2

        def trans(j, carry2):
            jrow = j * 128
            for gg in range(8):
                rows = _ROWS0[gg] + jrow
                for f in range(D_EDGE):
                    vals = attr_s[sslot, f // 8, j, f % 8, pl.ds(gg * 16, 16)]
                    plsc.store_scatter(attr_v.at[sslot], [rows, _COLS[f]], vals)
            return carry2

        lax.fori_loop(0, K, trans, 0)

    stage = stage_start(0)
    for d in stage:
        d.wait()
    stage = stage_start(1)
    transpose(0)
    descs_prev = []
    for s in range(STEPS):
        cur = s % 2
        descs = [
            pltpu.async_copy(
                attr_v.at[cur, pl.ds(j * ROW, ROW)],
                acc.at[idx_v.at[s % 3, j]],
                sem,
                add=True,
            )
            for j in range(K)
        ]
        if s + 1 < STEPS:
            for d in stage:
                d.wait()
        for d in descs_prev:
            d.wait()
        if s + 1 < STEPS:
            if s + 2 < STEPS:
                stage = stage_start(s + 2)
            transpose(s + 1)
        descs_prev = descs
    for d in descs_prev:
        d.wait()

    @pl.when(wid < REM_CHUNKS)
    def _rem_chunk():
        c = CPT * NW + wid
        pltpu.sync_copy(idx_hbm.at[pl.ds(c, 1), 1], idx_v.at[0, pl.ds(0, 1)])
        pltpu.sync_copy(attr4_hbm.at[:, pl.ds(c, 1)], attr_r)
        for gg in range(8):
            for f in range(D_EDGE):
                vals = attr_r[f // 8, 0, f % 8, pl.ds(gg * 16, 16)]
                plsc.store_scatter(attr_v.at[0], [_ROWS0[gg], _COLS[f]], vals)
        pltpu.sync_copy(
            attr_v.at[0, pl.ds(0, ROW)],
            acc.at[idx_v.at[0, 0]],
            add=True,
        )

    plsc.subcore_barrier()

    pltpu.sync_copy(acc.at[stripe], out_hbm.at[cid, stripe])


_sc_scatter = functools.partial(
    pl.kernel,
    mesh=plsc.VectorSubcoreMesh(core_axis_name="c", subcore_axis_name="s"),
    out_type=jax.ShapeDtypeStruct((NC, N_PAD, D_EDGE), jnp.float32),
    scratch_types=[
        pltpu.VMEM((3, K, ROW), jnp.int32),
        pltpu.VMEM((2, 2, K, 8, ROW), jnp.float32),
        pltpu.VMEM((2, 1, 8, ROW), jnp.float32),
        pltpu.VMEM((2, CHUNK, D_EDGE), jnp.float32),
        pltpu.VMEM_SHARED((N_PAD, D_EDGE), jnp.float32),
        pltpu.SemaphoreType.DMA,
        pltpu.SemaphoreType.DMA,
    ],
    compiler_params=pltpu.CompilerParams(
        use_tc_tiling_on_sc=False, needs_layout_passes=False
    ),
)(_sc_body)


def _combine_body(p_ref, o_ref):
    o_ref[...] = p_ref[0] + p_ref[1]


@jax.jit
def kernel(x, edge_index, edge_attr):
    del x
    attr4 = edge_attr.T.reshape(2, 8, N_CHUNKS, ROW).transpose(0, 2, 1, 3)
    idx3 = edge_index.astype(jnp.int32).reshape(2, N_CHUNKS, ROW).transpose(1, 0, 2)
    zeros = jnp.zeros((N_PAD, D_EDGE), jnp.float32)
    partials = _sc_scatter(zeros, idx3, attr4)
    wide = N_PAD * D_EDGE // 128
    combined = pl.pallas_call(
        _combine_body,
        out_shape=jax.ShapeDtypeStruct((wide, 128), jnp.float32),
    )(partials.reshape(NC, wide, 128))
    return combined.reshape(N_PAD, D_EDGE)[:N_NODES]
